# baseline (device time: 15941 ns/iter reference)
import jax
import jax.numpy as jnp
from jax import lax
from jax.experimental import pallas as pl
from jax.experimental.pallas import tpu as pltpu

NZ = 4
NCOL = 8


def kernel(x):
    _, m, n_total = x.shape
    n_chunk = n_total // NZ
    mb = m // NCOL

    def body(x_ref, out_ref, zrecv, zsend_sems, zrecv_sems,
             agsend_sems, agrecv_sems, agready_sem):
        my_x = lax.axis_index("x")
        my_y = lax.axis_index("y")
        my_z = lax.axis_index("z")
        my_blk = my_x * 4 + my_y
        row0 = my_blk * mb

        barrier_sem = pltpu.get_barrier_semaphore()
        for off in range(1, NZ):
            pl.semaphore_signal(
                barrier_sem, inc=1,
                device_id=(my_x, my_y, (my_z + off) % NZ),
                device_id_type=pl.DeviceIdType.MESH,
            )
        for off in range(1, NCOL):
            pb = (my_blk + off) % NCOL
            pl.semaphore_signal(
                agready_sem, inc=1,
                device_id=(pb // 4, pb % 4, my_z),
                device_id_type=pl.DeviceIdType.MESH,
            )
        pl.semaphore_wait(barrier_sem, NZ - 1)

        def piece(c):
            return x_ref.at[0, pl.ds(row0, mb), pl.ds(c * n_chunk, n_chunk)]

        zsends = []
        for off in range(1, NZ):
            tz = (my_z + off) % NZ
            s = pltpu.make_async_remote_copy(
                src_ref=piece(tz),
                dst_ref=zrecv.at[off - 1],
                send_sem=zsend_sems.at[off - 1],
                recv_sem=zrecv_sems.at[off - 1],
                device_id=(my_x, my_y, tz),
                device_id_type=pl.DeviceIdType.MESH,
            )
            s.start()
            zsends.append(s)

        acc = piece(my_z)[:, :]
        for off in range(1, NZ):
            zsends[off - 1].wait_recv()
            acc = acc + zrecv[off - 1]

        out_ref[pl.ds(row0, mb), :] = acc

        pl.semaphore_wait(agready_sem, NCOL - 1)
        agsends = {}
        for off in (3, 7, 2, 6, 1, 5, 4):
            pb = (my_blk + off) % NCOL
            s = pltpu.make_async_remote_copy(
                src_ref=out_ref.at[pl.ds(row0, mb), :],
                dst_ref=out_ref.at[pl.ds(row0, mb), :],
                send_sem=agsend_sems.at[off - 1],
                recv_sem=agrecv_sems.at[off - 1],
                device_id=(pb // 4, pb % 4, my_z),
                device_id_type=pl.DeviceIdType.MESH,
            )
            s.start()
            agsends[off] = s

        for off in (3, 7, 2, 6, 1, 5, 4):
            agsends[off].wait_recv()
        for off in range(1, NZ):
            zsends[off - 1].wait_send()
        for off in (3, 7, 2, 6, 1, 5, 4):
            agsends[off].wait_send()

    return pl.pallas_call(
        body,
        out_shape=jax.ShapeDtypeStruct((m, n_chunk), jnp.float32),
        in_specs=[pl.BlockSpec(memory_space=pltpu.VMEM)],
        out_specs=pl.BlockSpec(memory_space=pltpu.VMEM),
        scratch_shapes=[
            pltpu.VMEM((NZ - 1, mb, n_chunk), jnp.float32),
            pltpu.SemaphoreType.DMA((NZ - 1,)),
            pltpu.SemaphoreType.DMA((NZ - 1,)),
            pltpu.SemaphoreType.DMA((NCOL - 1,)),
            pltpu.SemaphoreType.DMA((NCOL - 1,)),
            pltpu.SemaphoreType.REGULAR,
        ],
        compiler_params=pltpu.CompilerParams(collective_id=0),
    )(x)


# device time: 14674 ns/iter; 1.0863x vs baseline; 1.0863x over previous
import jax
import jax.numpy as jnp
from jax import lax
from jax.experimental import pallas as pl
from jax.experimental.pallas import tpu as pltpu

NZ = 4
NCOL = 8


def kernel(x):
    _, m, n_total = x.shape
    n_chunk = n_total // NZ
    mb = m // NCOL

    def body(x_ref, out_ref, zrecv, zsend_sems, zrecv_sems,
             agsend_sems, agrecv_sems, agready_sem):
        my_x = lax.axis_index("x")
        my_y = lax.axis_index("y")
        my_z = lax.axis_index("z")
        my_blk = my_x * 4 + my_y
        row0 = my_blk * mb

        barrier_sem = pltpu.get_barrier_semaphore()
        for off in range(1, NZ):
            pl.semaphore_signal(
                barrier_sem, inc=1,
                device_id=(my_x, my_y, (my_z + off) % NZ),
                device_id_type=pl.DeviceIdType.MESH,
            )
        for off in range(1, NCOL):
            pb = (my_blk + off) % NCOL
            pl.semaphore_signal(
                agready_sem, inc=1,
                device_id=(pb // 4, pb % 4, my_z),
                device_id_type=pl.DeviceIdType.MESH,
            )
        pl.semaphore_wait(barrier_sem, NZ - 1)

        def piece(c):
            return x_ref.at[0, pl.ds(row0, mb), pl.ds(c * n_chunk, n_chunk)]

        zsends = []
        for off in range(1, NZ):
            tz = (my_z + off) % NZ
            s = pltpu.make_async_remote_copy(
                src_ref=piece(tz),
                dst_ref=zrecv.at[off - 1],
                send_sem=zsend_sems.at[off - 1],
                recv_sem=zrecv_sems.at[off - 1],
                device_id=(my_x, my_y, tz),
                device_id_type=pl.DeviceIdType.MESH,
            )
            s.start()
            zsends.append(s)

        acc = piece(my_z)[:, :]
        for off in range(1, NZ):
            zsends[off - 1].wait_recv()
            acc = acc + zrecv[off - 1]

        out_ref[pl.ds(row0, mb), :] = acc

        pl.semaphore_wait(agready_sem, NCOL - 1)
        agsends = {}
        for off in range(1, NCOL):
            pb = (my_blk + off) % NCOL
            s = pltpu.make_async_remote_copy(
                src_ref=out_ref.at[pl.ds(row0, mb), :],
                dst_ref=out_ref.at[pl.ds(row0, mb), :],
                send_sem=agsend_sems.at[off - 1],
                recv_sem=agrecv_sems.at[off - 1],
                device_id=(pb // 4, pb % 4, my_z),
                device_id_type=pl.DeviceIdType.MESH,
            )
            s.start()
            agsends[off] = s

        for off in range(1, NCOL):
            agsends[off].wait_recv()
        for off in range(1, NZ):
            zsends[off - 1].wait_send()
        for off in range(1, NCOL):
            agsends[off].wait_send()

    return pl.pallas_call(
        body,
        out_shape=jax.ShapeDtypeStruct((m, n_chunk), jnp.float32),
        in_specs=[pl.BlockSpec(memory_space=pltpu.VMEM)],
        out_specs=pl.BlockSpec(memory_space=pltpu.VMEM),
        scratch_shapes=[
            pltpu.VMEM((NZ - 1, mb, n_chunk), jnp.float32),
            pltpu.SemaphoreType.DMA((NZ - 1,)),
            pltpu.SemaphoreType.DMA((NZ - 1,)),
            pltpu.SemaphoreType.DMA((NCOL - 1,)),
            pltpu.SemaphoreType.DMA((NCOL - 1,)),
            pltpu.SemaphoreType.REGULAR,
        ],
        compiler_params=pltpu.CompilerParams(collective_id=0),
    )(x)


# device time: 14562 ns/iter; 1.0947x vs baseline; 1.0077x over previous
import jax
import jax.numpy as jnp
from jax import lax
from jax.experimental import pallas as pl
from jax.experimental.pallas import tpu as pltpu

NZ = 4
NCOL = 8


def kernel(x):
    _, m, n_total = x.shape
    n_chunk = n_total // NZ
    mb = m // NCOL

    def body(x_ref, out_ref, zrecv, zsend_sems, zrecv_sems,
             agsend_sems, agrecv_sems, agready_sem, zready_sems):
        my_x = lax.axis_index("x")
        my_y = lax.axis_index("y")
        my_z = lax.axis_index("z")
        my_blk = my_x * 4 + my_y
        row0 = my_blk * mb

        barrier_sem = pltpu.get_barrier_semaphore()
        for off in range(1, NZ - 1):
            pl.semaphore_signal(
                zready_sems.at[off - 1], inc=1,
                device_id=(my_x, my_y, (my_z - off) % NZ),
                device_id_type=pl.DeviceIdType.MESH,
            )
        pl.semaphore_signal(
            barrier_sem, inc=1,
            device_id=(my_x, my_y, (my_z - (NZ - 1)) % NZ),
            device_id_type=pl.DeviceIdType.MESH,
        )
        for off in range(1, NCOL):
            pb = (my_blk + off) % NCOL
            pl.semaphore_signal(
                agready_sem, inc=1,
                device_id=(pb // 4, pb % 4, my_z),
                device_id_type=pl.DeviceIdType.MESH,
            )

        def piece(c):
            return x_ref.at[0, pl.ds(row0, mb), pl.ds(c * n_chunk, n_chunk)]

        zsends = []
        for off in range(1, NZ):
            if off < NZ - 1:
                pl.semaphore_wait(zready_sems.at[off - 1], 1)
            else:
                pl.semaphore_wait(barrier_sem, 1)
            tz = (my_z + off) % NZ
            s = pltpu.make_async_remote_copy(
                src_ref=piece(tz),
                dst_ref=zrecv.at[off - 1],
                send_sem=zsend_sems.at[off - 1],
                recv_sem=zrecv_sems.at[off - 1],
                device_id=(my_x, my_y, tz),
                device_id_type=pl.DeviceIdType.MESH,
            )
            s.start()
            zsends.append(s)

        acc = piece(my_z)[:, :]
        for off in range(1, NZ):
            zsends[off - 1].wait_recv()
            acc = acc + zrecv[off - 1]

        out_ref[pl.ds(row0, mb), :] = acc

        pl.semaphore_wait(agready_sem, NCOL - 1)
        agsends = {}
        for off in range(1, NCOL):
            pb = (my_blk + off) % NCOL
            s = pltpu.make_async_remote_copy(
                src_ref=out_ref.at[pl.ds(row0, mb), :],
                dst_ref=out_ref.at[pl.ds(row0, mb), :],
                send_sem=agsend_sems.at[off - 1],
                recv_sem=agrecv_sems.at[off - 1],
                device_id=(pb // 4, pb % 4, my_z),
                device_id_type=pl.DeviceIdType.MESH,
            )
            s.start()
            agsends[off] = s

        for off in range(1, NCOL):
            agsends[off].wait_recv()
        for off in range(1, NZ):
            zsends[off - 1].wait_send()
        for off in range(1, NCOL):
            agsends[off].wait_send()

    return pl.pallas_call(
        body,
        out_shape=jax.ShapeDtypeStruct((m, n_chunk), jnp.float32),
        in_specs=[pl.BlockSpec(memory_space=pltpu.VMEM)],
        out_specs=pl.BlockSpec(memory_space=pltpu.VMEM),
        scratch_shapes=[
            pltpu.VMEM((NZ - 1, mb, n_chunk), jnp.float32),
            pltpu.SemaphoreType.DMA((NZ - 1,)),
            pltpu.SemaphoreType.DMA((NZ - 1,)),
            pltpu.SemaphoreType.DMA((NCOL - 1,)),
            pltpu.SemaphoreType.DMA((NCOL - 1,)),
            pltpu.SemaphoreType.REGULAR,
            pltpu.SemaphoreType.REGULAR((NZ - 2,)),
        ],
        compiler_params=pltpu.CompilerParams(collective_id=0),
    )(x)
